# baseline (speedup 1.0000x reference)
import functools

import jax
import jax.numpy as jnp
from jax import lax
from jax.experimental import pallas as pl
from jax.experimental.pallas import tpu as pltpu
from jax.experimental.pallas import tpu_sc as plsc

_S = 64.0
_M = 0.5


def _sc_fix_vals(cosine, label, cms, sms):
    B, C = cosine.shape
    info = plsc.get_sparse_core_info()
    NC, NS, L = info.num_cores, info.num_subcores, info.num_lanes
    NW = NC * NS
    rpw = B // NW
    mesh = plsc.VectorSubcoreMesh(core_axis_name="c", subcore_axis_name="s")

    @functools.partial(
        pl.kernel,
        mesh=mesh,
        compiler_params=pltpu.CompilerParams(needs_layout_passes=False),
        out_type=jax.ShapeDtypeStruct((B,), jnp.float32),
        scratch_types=[
            pltpu.VMEM((rpw,), jnp.int32),
            pltpu.VMEM((rpw, 8, 128), jnp.float32),
            pltpu.VMEM((rpw,), jnp.float32),
            pltpu.VMEM((rpw,), jnp.float32),
            pltpu.VMEM((rpw,), jnp.float32),
            pltpu.SemaphoreType.DMA,
        ],
    )
    def k(cos_hbm, lab_hbm, cms_hbm, sms_hbm, out_hbm,
          lab_v, win_v, cm_v, sm_v, fix_v, sem):
        wid = lax.axis_index("s") * NC + lax.axis_index("c")
        base = wid * rpw
        pltpu.sync_copy(lab_hbm.at[pl.ds(base, rpw)], lab_v)
        pltpu.sync_copy(cms_hbm.at[pl.ds(base, rpw)], cm_v)
        pltpu.sync_copy(sms_hbm.at[pl.ds(base, rpw)], sm_v)
        lanes = lax.iota(jnp.int32, L)
        cps = []
        for j in range(rpw):
            grp = lab_v[pl.ds((j // L) * L, L)]
            c = jnp.sum(jnp.where(lanes == (j % L), grp, 0))
            cst = pl.multiple_of((c >> 7) << 7, 128)
            rst = pl.multiple_of(((base + j) >> 3) << 3, 8)
            cp = pltpu.make_async_copy(
                cos_hbm.at[pl.ds(rst, 8), pl.ds(cst, 128)], win_v.at[j], sem)
            cp.start()
            cps.append(cp)
        for cp in cps:
            cp.wait()
        for g in range(rpw // L):
            sl = pl.ds(g * L, L)
            lab = lab_v[sl]
            coff = lab & 127
            rows = lax.iota(jnp.int32, L) + g * L
            roff = (rows + base) & 7
            x = plsc.load_gather(win_v, [rows, roff, coff])
            x = jnp.minimum(jnp.maximum(x, -1.0), 1.0)
            a = jnp.maximum(1.0 - x * x, 0.0)
            y = 0.27 + 0.77 * a
            for _ in range(4):
                y = 0.5 * (y + a / y)
            fix_v[sl] = x * cm_v[sl] - y * sm_v[sl]
        pltpu.sync_copy(fix_v, out_hbm.at[pl.ds(base, rpw)])

    return k(cosine, label, cms, sms)


def _tc_apply(cosine, label2d, fix2d, bB=8, K=4):
    B, C = cosine.shape
    rows_per_step = bB * K
    nsteps = B // rows_per_step

    def body(cos_hbm, lab_ref, fix_ref, out_hbm,
             bufs_in, bufs_out, in_sems, out_sems):
        g = pl.program_id(0)

        @pl.when(g == 0)
        def _prime():
            for k in range(K):
                pltpu.make_async_copy(
                    cos_hbm.at[pl.ds(k * bB, bB)], bufs_in.at[k],
                    in_sems.at[k]).start()

        for k in range(K):
            chunk = g * K + k
            row0 = chunk * bB
            pltpu.make_async_copy(
                cos_hbm.at[pl.ds(row0, bB)], bufs_in.at[k],
                in_sems.at[k]).wait()

            @pl.when(g > 0)
            def _drain_out():
                pltpu.make_async_copy(
                    bufs_out.at[k], out_hbm.at[pl.ds(row0 - rows_per_step, bB)],
                    out_sems.at[k]).wait()

            x = bufs_in[k]
            lab = lab_ref[pl.ds(k * bB, bB), :]
            fv = fix_ref[pl.ds(k * bB, bB), :]
            cols = lax.broadcasted_iota(jnp.int32, (bB, C), 1)
            bufs_out[k] = jnp.where(cols == lab, fv, x * _S)
            pltpu.make_async_copy(
                bufs_out.at[k], out_hbm.at[pl.ds(row0, bB)],
                out_sems.at[k]).start()

            @pl.when(g + 1 < nsteps)
            def _prefetch():
                pltpu.make_async_copy(
                    cos_hbm.at[pl.ds(row0 + rows_per_step, bB)],
                    bufs_in.at[k], in_sems.at[k]).start()

        @pl.when(g == nsteps - 1)
        def _drain_all():
            for k in range(K):
                pltpu.make_async_copy(
                    bufs_out.at[k],
                    out_hbm.at[pl.ds((g * K + k) * bB, bB)],
                    out_sems.at[k]).wait()

    return pl.pallas_call(
        body,
        grid=(nsteps,),
        in_specs=[
            pl.BlockSpec(memory_space=pl.ANY),
            pl.BlockSpec((rows_per_step, 1), lambda i: (i, 0)),
            pl.BlockSpec((rows_per_step, 1), lambda i: (i, 0)),
        ],
        out_specs=pl.BlockSpec(memory_space=pl.ANY),
        out_shape=jax.ShapeDtypeStruct((B, C), jnp.float32),
        scratch_shapes=[
            pltpu.VMEM((K, bB, C), jnp.float32),
            pltpu.VMEM((K, bB, C), jnp.float32),
            pltpu.SemaphoreType.DMA((K,)),
            pltpu.SemaphoreType.DMA((K,)),
        ],
    )(cosine, label2d, fix2d)


def kernel(cosine, label):
    B, C = cosine.shape
    margin = jax.random.normal(jax.random.key(42), (B,), jnp.float32) * 0.1 + _M
    cms = jnp.cos(margin) * _S
    sms = jnp.sin(margin) * _S
    fix = _sc_fix_vals(cosine, label, cms, sms)
    return _tc_apply(cosine, label.reshape(B, 1), fix.reshape(B, 1))


# speedup vs baseline: 1.0005x; 1.0005x over previous
import functools

import jax
import jax.numpy as jnp
from jax import lax
from jax.experimental import pallas as pl
from jax.experimental.pallas import tpu as pltpu
from jax.experimental.pallas import tpu_sc as plsc

_S = 64.0
_M = 0.5


def _sc_fix_vals(cosine, label, cms, sms):
    B, C = cosine.shape
    info = plsc.get_sparse_core_info()
    NC, NS, L = info.num_cores, info.num_subcores, info.num_lanes
    NW = NC * NS
    rpw = B // NW
    mesh = plsc.VectorSubcoreMesh(core_axis_name="c", subcore_axis_name="s")

    @functools.partial(
        pl.kernel,
        mesh=mesh,
        compiler_params=pltpu.CompilerParams(needs_layout_passes=False),
        out_type=jax.ShapeDtypeStruct((B,), jnp.float32),
        scratch_types=[
            pltpu.VMEM((rpw,), jnp.int32),
            pltpu.VMEM((rpw, 8, 128), jnp.float32),
            pltpu.VMEM((rpw,), jnp.float32),
            pltpu.VMEM((rpw,), jnp.float32),
            pltpu.VMEM((rpw,), jnp.float32),
            pltpu.SemaphoreType.DMA,
        ],
    )
    def k(cos_hbm, lab_hbm, cms_hbm, sms_hbm, out_hbm,
          lab_v, win_v, cm_v, sm_v, fix_v, sem):
        wid = lax.axis_index("s") * NC + lax.axis_index("c")
        base = wid * rpw
        pltpu.sync_copy(lab_hbm.at[pl.ds(base, rpw)], lab_v)
        pltpu.sync_copy(cms_hbm.at[pl.ds(base, rpw)], cm_v)
        pltpu.sync_copy(sms_hbm.at[pl.ds(base, rpw)], sm_v)
        lanes = lax.iota(jnp.int32, L)
        cps = []
        for j in range(rpw):
            grp = lab_v[pl.ds((j // L) * L, L)]
            c = jnp.sum(jnp.where(lanes == (j % L), grp, 0))
            cst = pl.multiple_of((c >> 7) << 7, 128)
            rst = pl.multiple_of(((base + j) >> 3) << 3, 8)
            cp = pltpu.make_async_copy(
                cos_hbm.at[pl.ds(rst, 8), pl.ds(cst, 128)], win_v.at[j], sem)
            cp.start()
            cps.append(cp)
        for cp in cps:
            cp.wait()
        for g in range(rpw // L):
            sl = pl.ds(g * L, L)
            lab = lab_v[sl]
            coff = lab & 127
            rows = lax.iota(jnp.int32, L) + g * L
            roff = (rows + base) & 7
            x = plsc.load_gather(win_v, [rows, roff, coff])
            x = jnp.minimum(jnp.maximum(x, -1.0), 1.0)
            a = jnp.maximum(1.0 - x * x, 0.0)
            y = 0.27 + 0.77 * a
            for _ in range(4):
                y = 0.5 * (y + a / y)
            fix_v[sl] = x * cm_v[sl] - y * sm_v[sl]
        pltpu.sync_copy(fix_v, out_hbm.at[pl.ds(base, rpw)])

    return k(cosine, label, cms, sms)


def _tc_apply(cosine, label2d, fix2d, bB=16, K=2, S=2):
    B, C = cosine.shape
    sB = bB // S
    rows_per_step = bB * K
    nsteps = B // rows_per_step

    def _stripes(hbm, bufs, sems, row0, k, direction):
        cps = []
        for r in range(S):
            src = hbm.at[pl.ds(row0 + r * sB, sB)]
            dst = bufs.at[k, pl.ds(r * sB, sB)]
            if direction == "out":
                src, dst = dst, src
            cps.append(pltpu.make_async_copy(src, dst, sems.at[k, r]))
        return cps

    def body(cos_hbm, lab_ref, fix_ref, out_hbm,
             bufs_in, bufs_out, in_sems, out_sems):
        g = pl.program_id(0)

        @pl.when(g == 0)
        def _prime():
            for k in range(K):
                for cp in _stripes(cos_hbm, bufs_in, in_sems, k * bB, k, "in"):
                    cp.start()

        for k in range(K):
            row0 = (g * K + k) * bB
            for cp in _stripes(cos_hbm, bufs_in, in_sems, row0, k, "in"):
                cp.wait()

            @pl.when(g > 0)
            def _drain_out():
                for cp in _stripes(out_hbm, bufs_out, out_sems,
                                   row0 - rows_per_step, k, "out"):
                    cp.wait()

            x = bufs_in[k]
            lab = lab_ref[pl.ds(k * bB, bB), :]
            fv = fix_ref[pl.ds(k * bB, bB), :]
            cols = lax.broadcasted_iota(jnp.int32, (1, C), 1)
            bufs_out[k] = jnp.where(cols == lab, fv, x * _S)
            for cp in _stripes(out_hbm, bufs_out, out_sems, row0, k, "out"):
                cp.start()

            @pl.when(g + 1 < nsteps)
            def _prefetch():
                for cp in _stripes(cos_hbm, bufs_in, in_sems,
                                   row0 + rows_per_step, k, "in"):
                    cp.start()

        @pl.when(g == nsteps - 1)
        def _drain_all():
            for k in range(K):
                for cp in _stripes(out_hbm, bufs_out, out_sems,
                                   (g * K + k) * bB, k, "out"):
                    cp.wait()

    return pl.pallas_call(
        body,
        grid=(nsteps,),
        in_specs=[
            pl.BlockSpec(memory_space=pl.ANY),
            pl.BlockSpec((rows_per_step, 1), lambda i: (i, 0)),
            pl.BlockSpec((rows_per_step, 1), lambda i: (i, 0)),
        ],
        out_specs=pl.BlockSpec(memory_space=pl.ANY),
        out_shape=jax.ShapeDtypeStruct((B, C), jnp.float32),
        scratch_shapes=[
            pltpu.VMEM((K, bB, C), jnp.float32),
            pltpu.VMEM((K, bB, C), jnp.float32),
            pltpu.SemaphoreType.DMA((K, S)),
            pltpu.SemaphoreType.DMA((K, S)),
        ],
    )(cosine, label2d, fix2d)


def kernel(cosine, label):
    B, C = cosine.shape
    margin = jax.random.normal(jax.random.key(42), (B,), jnp.float32) * 0.1 + _M
    cms = jnp.cos(margin) * _S
    sms = jnp.sin(margin) * _S
    fix = _sc_fix_vals(cosine, label, cms, sms)
    return _tc_apply(cosine, label.reshape(B, 1), fix.reshape(B, 1))
